# baseline (speedup 1.0000x reference)
import functools

import jax
import jax.numpy as jnp
from jax import lax
from jax.experimental import pallas as pl
from jax.experimental.pallas import tpu as pltpu
from jax.experimental.pallas import tpu_sc as plsc

_NUM_ENTITIES = 256
_NUM_RELATIONS = 64
_EMBED_DIM = 128
_BATCH = 16384
_HIST = 20

_NC = 2
_NS = 16
_L = 16
_NW = _NC * _NS

_EPW = _BATCH * _HIST // _NW
_BPW = _BATCH // _NW
_RPW = _BATCH // _NW
_WB = 16
_CL = _WB * _HIST
_E_CHUNKS = _BPW // _WB
_E_GROUPS = _E_CHUNKS // 2
_RCH = 64
_R_CHUNKS = _RPW // _RCH
_R_GROUPS = _R_CHUNKS // 2
_NK = _EMBED_DIM // _L

_mesh = plsc.VectorSubcoreMesh(core_axis_name="c", subcore_axis_name="s")


@functools.partial(
    pl.kernel,
    mesh=_mesh,
    compiler_params=pltpu.CompilerParams(needs_layout_passes=False),
    out_type=(
        jax.ShapeDtypeStruct((_HIST, _BATCH, _EMBED_DIM), jnp.float32),
        jax.ShapeDtypeStruct((_BATCH, _EMBED_DIM), jnp.float32),
    ),
    scratch_types=[
        pltpu.VMEM((_HIST * _BPW,), jnp.int32),
        pltpu.VMEM((_RPW,), jnp.int32),
        pltpu.VMEM((_NUM_ENTITIES,), jnp.int32),
        pltpu.VMEM((_NUM_RELATIONS,), jnp.int32),
        pltpu.VMEM((64 * _EMBED_DIM,), jnp.float32),
        pltpu.VMEM((16 * _EMBED_DIM,), jnp.float32),
        [pltpu.VMEM((_CL, _EMBED_DIM), jnp.float32) for _ in range(2)],
        [pltpu.VMEM((_RCH, _EMBED_DIM), jnp.float32) for _ in range(2)],
        [pltpu.SemaphoreType.DMA for _ in range(2)],
        [pltpu.SemaphoreType.DMA for _ in range(2)],
    ],
)
def _cluster_embed_sc(eids_t, rids, elab, rlab, etab_flat, rtab_flat,
                      out_e, out_r,
                      ids_v, rids_v, elab_v, rlab_v, etab_v, rtab_v,
                      rows_v, rrows_v, w_sem, rw_sem):
    wid = lax.axis_index("s") * _NC + lax.axis_index("c")
    bbase = wid * _BPW
    rbase = wid * _RPW

    for h in range(_HIST):
        pltpu.sync_copy(eids_t.at[h].at[pl.ds(bbase, _BPW)],
                        ids_v.at[pl.ds(h * _BPW, _BPW)])
    pltpu.sync_copy(rids.at[pl.ds(rbase, _RPW)], rids_v)
    pltpu.sync_copy(elab, elab_v)
    pltpu.sync_copy(rlab, rlab_v)
    pltpu.sync_copy(etab_flat, etab_v)
    pltpu.sync_copy(rtab_flat, rtab_v)

    def _replicate16(ids16, lab_ref, tab_ref, dst_ref, row0):
        cl16 = plsc.load_gather(lab_ref, [ids16])
        bases = [cl16[l] * _EMBED_DIM for l in range(_L)]
        vals = [tab_ref[pl.ds(bases[l], _L)] for l in range(_L)]
        for k in range(_NK):
            cur = vals
            if k + 1 < _NK:
                vals = [tab_ref[pl.ds(bases[l] + (k + 1) * _L, _L)]
                        for l in range(_L)]
            for l in range(_L):
                dst_ref[row0 + l, pl.ds(k * _L, _L)] = cur[l]

    def _e_wait(b):
        for h in range(_HIST):
            pltpu.make_async_copy(
                rows_v[b].at[pl.ds(h * _WB, _WB)],
                out_e.at[h].at[pl.ds(bbase, _WB)], w_sem[b]).wait()

    def _egroup(g, carry):
        for b in range(2):
            c = g * 2 + b

            @pl.when(c >= 2)
            def _():
                _e_wait(b)

            @plsc.parallel_loop(0, _HIST, unroll=10)
            def _(h):
                ids16 = ids_v[pl.ds(h * _BPW + c * _WB, _L)]
                _replicate16(ids16, elab_v, etab_v, rows_v[b], h * _WB)

            for h in range(_HIST):
                pltpu.async_copy(
                    rows_v[b].at[pl.ds(h * _WB, _WB)],
                    out_e.at[h].at[pl.ds(bbase + c * _WB, _WB)], w_sem[b])
        return carry

    lax.fori_loop(0, _E_GROUPS, _egroup, 0)
    for b in range(2):
        _e_wait(b)

    def _r_wait(b):
        pltpu.make_async_copy(
            rrows_v[b], out_r.at[pl.ds(rbase, _RCH)], rw_sem[b]).wait()

    def _rgroup(g, carry):
        for b in range(2):
            c = g * 2 + b

            @pl.when(c >= 2)
            def _():
                _r_wait(b)

            @plsc.parallel_loop(0, _RCH // _L, unroll=2)
            def _(grp):
                ids16 = rids_v[pl.ds(c * _RCH + grp * _L, _L)]
                _replicate16(ids16, rlab_v, rtab_v, rrows_v[b], grp * _L)

            pltpu.async_copy(
                rrows_v[b], out_r.at[pl.ds(rbase + c * _RCH, _RCH)], rw_sem[b])
        return carry

    lax.fori_loop(0, _R_GROUPS, _rgroup, 0)
    for b in range(2):
        _r_wait(b)


def kernel(entity_ids, relation_ids, entity_cluster_labels,
           relation_cluster_labels, entity_table, relation_table):
    ent_t, rel_emb = _cluster_embed_sc(
        entity_ids.T,
        relation_ids,
        entity_cluster_labels,
        relation_cluster_labels,
        entity_table.reshape(-1),
        relation_table.reshape(-1),
    )
    return jnp.transpose(ent_t, (1, 0, 2)), rel_emb


# speedup vs baseline: 1.8144x; 1.8144x over previous
import functools

import jax
import jax.numpy as jnp
from jax import lax
from jax.experimental import pallas as pl
from jax.experimental.pallas import tpu as pltpu
from jax.experimental.pallas import tpu_sc as plsc

_NUM_ENTITIES = 256
_NUM_RELATIONS = 64
_EMBED_DIM = 128
_BATCH = 16384
_HIST = 20

_NC = 2
_NS = 16
_L = 16
_NW = _NC * _NS

_EPW = _BATCH * _HIST // _NW
_BPW = _BATCH // _NW
_RPW = _BATCH // _NW
_WB = 16
_CL = _WB * _HIST
_E_CHUNKS = _BPW // _WB
_E_GROUPS = _E_CHUNKS // 2
_RCH = 64
_R_CHUNKS = _RPW // _RCH
_R_GROUPS = _R_CHUNKS // 2
_NK = _EMBED_DIM // _L

_mesh = plsc.VectorSubcoreMesh(core_axis_name="c", subcore_axis_name="s")


@functools.partial(
    pl.kernel,
    mesh=_mesh,
    compiler_params=pltpu.CompilerParams(needs_layout_passes=False),
    out_type=(
        jax.ShapeDtypeStruct((_HIST, _BATCH, _EMBED_DIM), jnp.float32),
        jax.ShapeDtypeStruct((_BATCH, _EMBED_DIM), jnp.float32),
    ),
    scratch_types=[
        pltpu.VMEM((_HIST * _BPW,), jnp.int32),
        pltpu.VMEM((_RPW,), jnp.int32),
        pltpu.VMEM((_NUM_ENTITIES,), jnp.int32),
        pltpu.VMEM((_NUM_RELATIONS,), jnp.int32),
        pltpu.VMEM((64 * _EMBED_DIM,), jnp.float32),
        pltpu.VMEM((16 * _EMBED_DIM,), jnp.float32),
        [pltpu.VMEM((_CL, _EMBED_DIM), jnp.float32) for _ in range(2)],
        [pltpu.VMEM((_RCH, _EMBED_DIM), jnp.float32) for _ in range(2)],
        [pltpu.SemaphoreType.DMA for _ in range(2)],
        [pltpu.SemaphoreType.DMA for _ in range(2)],
    ],
)
def _cluster_embed_sc(eids_t, rids, elab, rlab, etab_flat, rtab_flat,
                      out_e, out_r,
                      ids_v, rids_v, elab_v, rlab_v, etab_v, rtab_v,
                      rows_v, rrows_v, w_sem, rw_sem):
    wid = lax.axis_index("s") * _NC + lax.axis_index("c")
    bbase = wid * _BPW
    rbase = wid * _RPW

    for h in range(_HIST):
        pltpu.sync_copy(eids_t.at[h].at[pl.ds(bbase, _BPW)],
                        ids_v.at[pl.ds(h * _BPW, _BPW)])
    pltpu.sync_copy(rids.at[pl.ds(rbase, _RPW)], rids_v)
    pltpu.sync_copy(elab, elab_v)
    pltpu.sync_copy(rlab, rlab_v)
    pltpu.sync_copy(etab_flat, etab_v)
    pltpu.sync_copy(rtab_flat, rtab_v)

    def _replicate16(ids16, lab_ref, tab_ref, dst_ref, row0):
        cl16 = plsc.load_gather(lab_ref, [ids16])
        bases = [cl16[l] * _EMBED_DIM for l in range(_L)]
        vals = [tab_ref[pl.ds(bases[l], _L)] for l in range(_L)]
        for k in range(_NK):
            cur = vals
            if k + 1 < _NK:
                vals = [tab_ref[pl.ds(bases[l] + (k + 1) * _L, _L)]
                        for l in range(_L)]
            for l in range(_L):
                dst_ref[row0 + l, pl.ds(k * _L, _L)] = cur[l]

    def _e_wait(b):
        for h in range(_HIST):
            pltpu.make_async_copy(
                rows_v[b].at[pl.ds(h * _WB, _WB)],
                out_e.at[h].at[pl.ds(bbase, _WB)], w_sem[b]).wait()

    def _egroup(g, carry):
        for b in range(2):
            c = g * 2 + b

            @pl.when(c >= 2)
            def _():
                _e_wait(b)

            @plsc.parallel_loop(0, _HIST, unroll=5)
            def _(h):
                ids16 = ids_v[pl.ds(h * _BPW + c * _WB, _L)]
                _replicate16(ids16, elab_v, etab_v, rows_v[b], h * _WB)

            for h in range(_HIST):
                pltpu.async_copy(
                    rows_v[b].at[pl.ds(h * _WB, _WB)],
                    out_e.at[h].at[pl.ds(bbase + c * _WB, _WB)], w_sem[b])
        return carry

    lax.fori_loop(0, _E_GROUPS, _egroup, 0)
    for b in range(2):
        _e_wait(b)

    def _r_wait(b):
        pltpu.make_async_copy(
            rrows_v[b], out_r.at[pl.ds(rbase, _RCH)], rw_sem[b]).wait()

    def _rgroup(g, carry):
        for b in range(2):
            c = g * 2 + b

            @pl.when(c >= 2)
            def _():
                _r_wait(b)

            @plsc.parallel_loop(0, _RCH // _L, unroll=4)
            def _(grp):
                ids16 = rids_v[pl.ds(c * _RCH + grp * _L, _L)]
                _replicate16(ids16, rlab_v, rtab_v, rrows_v[b], grp * _L)

            pltpu.async_copy(
                rrows_v[b], out_r.at[pl.ds(rbase + c * _RCH, _RCH)], rw_sem[b])
        return carry

    lax.fori_loop(0, _R_GROUPS, _rgroup, 0)
    for b in range(2):
        _r_wait(b)


def kernel(entity_ids, relation_ids, entity_cluster_labels,
           relation_cluster_labels, entity_table, relation_table):
    ent_t, rel_emb = _cluster_embed_sc(
        entity_ids.T,
        relation_ids,
        entity_cluster_labels,
        relation_cluster_labels,
        entity_table.reshape(-1),
        relation_table.reshape(-1),
    )
    return jnp.transpose(ent_t, (1, 0, 2)), rel_emb


# speedup vs baseline: 1.9783x; 1.0904x over previous
import functools

import jax
import jax.numpy as jnp
from jax import lax
from jax.experimental import pallas as pl
from jax.experimental.pallas import tpu as pltpu
from jax.experimental.pallas import tpu_sc as plsc

_NUM_ENTITIES = 256
_NUM_RELATIONS = 64
_EMBED_DIM = 128
_BATCH = 16384
_HIST = 20

_NC = 2
_NS = 16
_L = 16
_NW = _NC * _NS

_EPW = _BATCH * _HIST // _NW
_BPW = _BATCH // _NW
_RPW = _BATCH // _NW
_WB = 16
_CL = _WB * _HIST
_E_CHUNKS = _BPW // _WB
_E_GROUPS = _E_CHUNKS // 2
_RCH = 64
_R_CHUNKS = _RPW // _RCH
_R_GROUPS = _R_CHUNKS // 2
_NK = _EMBED_DIM // _L

_mesh = plsc.VectorSubcoreMesh(core_axis_name="c", subcore_axis_name="s")


@functools.partial(
    pl.kernel,
    mesh=_mesh,
    compiler_params=pltpu.CompilerParams(needs_layout_passes=False),
    out_type=(
        jax.ShapeDtypeStruct((_HIST, _BATCH, _EMBED_DIM), jnp.float32),
        jax.ShapeDtypeStruct((_BATCH, _EMBED_DIM), jnp.float32),
    ),
    scratch_types=[
        pltpu.VMEM((_HIST * _BPW,), jnp.int32),
        pltpu.VMEM((_RPW,), jnp.int32),
        pltpu.VMEM((_NUM_ENTITIES,), jnp.int32),
        pltpu.VMEM((_NUM_RELATIONS,), jnp.int32),
        pltpu.VMEM((64 * _EMBED_DIM,), jnp.float32),
        pltpu.VMEM((16 * _EMBED_DIM,), jnp.float32),
        [pltpu.VMEM((_CL, _EMBED_DIM), jnp.float32) for _ in range(2)],
        [pltpu.VMEM((_RCH, _EMBED_DIM), jnp.float32) for _ in range(2)],
        [pltpu.SemaphoreType.DMA for _ in range(2)],
        [pltpu.SemaphoreType.DMA for _ in range(2)],
    ],
)
def _cluster_embed_sc(eids_t, rids, elab, rlab, etab_flat, rtab_flat,
                      out_e, out_r,
                      ids_v, rids_v, elab_v, rlab_v, etab_v, rtab_v,
                      rows_v, rrows_v, w_sem, rw_sem):
    wid = lax.axis_index("s") * _NC + lax.axis_index("c")
    bbase = wid * _BPW
    rbase = wid * _RPW

    for h in range(_HIST):
        pltpu.sync_copy(eids_t.at[h].at[pl.ds(bbase, _BPW)],
                        ids_v.at[pl.ds(h * _BPW, _BPW)])
    pltpu.sync_copy(rids.at[pl.ds(rbase, _RPW)], rids_v)
    pltpu.sync_copy(elab, elab_v)
    pltpu.sync_copy(rlab, rlab_v)
    pltpu.sync_copy(etab_flat, etab_v)
    pltpu.sync_copy(rtab_flat, rtab_v)

    def _replicate16(ids16, lab_ref, tab_ref, dst_ref, row0):
        cl16 = plsc.load_gather(lab_ref, [ids16])
        bases = [cl16[l] * _EMBED_DIM for l in range(_L)]
        vals = [tab_ref[pl.ds(bases[l], _L)] for l in range(_L)]
        for k in range(_NK):
            cur = vals
            if k + 1 < _NK:
                vals = [tab_ref[pl.ds(bases[l] + (k + 1) * _L, _L)]
                        for l in range(_L)]
            for l in range(_L):
                dst_ref[row0 + l, pl.ds(k * _L, _L)] = cur[l]

    def _e_wait(b):
        for h in range(_HIST):
            pltpu.make_async_copy(
                rows_v[b].at[pl.ds(h * _WB, _WB)],
                out_e.at[h].at[pl.ds(bbase, _WB)], w_sem[b]).wait()

    def _egroup(g, carry):
        for b in range(2):
            c = g * 2 + b

            @pl.when(c >= 2)
            def _():
                _e_wait(b)

            @plsc.parallel_loop(0, _HIST, unroll=4)
            def _(h):
                ids16 = ids_v[pl.ds(h * _BPW + c * _WB, _L)]
                _replicate16(ids16, elab_v, etab_v, rows_v[b], h * _WB)

            for h in range(_HIST):
                pltpu.async_copy(
                    rows_v[b].at[pl.ds(h * _WB, _WB)],
                    out_e.at[h].at[pl.ds(bbase + c * _WB, _WB)], w_sem[b])
        return carry

    lax.fori_loop(0, _E_GROUPS, _egroup, 0)
    for b in range(2):
        _e_wait(b)

    def _r_wait(b):
        pltpu.make_async_copy(
            rrows_v[b], out_r.at[pl.ds(rbase, _RCH)], rw_sem[b]).wait()

    def _rgroup(g, carry):
        for b in range(2):
            c = g * 2 + b

            @pl.when(c >= 2)
            def _():
                _r_wait(b)

            @plsc.parallel_loop(0, _RCH // _L, unroll=4)
            def _(grp):
                ids16 = rids_v[pl.ds(c * _RCH + grp * _L, _L)]
                _replicate16(ids16, rlab_v, rtab_v, rrows_v[b], grp * _L)

            pltpu.async_copy(
                rrows_v[b], out_r.at[pl.ds(rbase + c * _RCH, _RCH)], rw_sem[b])
        return carry

    lax.fori_loop(0, _R_GROUPS, _rgroup, 0)
    for b in range(2):
        _r_wait(b)


def kernel(entity_ids, relation_ids, entity_cluster_labels,
           relation_cluster_labels, entity_table, relation_table):
    ent_t, rel_emb = _cluster_embed_sc(
        entity_ids.T,
        relation_ids,
        entity_cluster_labels,
        relation_cluster_labels,
        entity_table.reshape(-1),
        relation_table.reshape(-1),
    )
    return jnp.transpose(ent_t, (1, 0, 2)), rel_emb
